# baseline (speedup 1.0000x reference)
import functools

import jax
import jax.numpy as jnp
from jax import lax
from jax.experimental import pallas as pl
from jax.experimental.pallas import tpu as pltpu
from jax.experimental.pallas import tpu_sc as plsc

EMBED = 64
BATCH = 16384
HIDDEN = 128

NC = 2
NS = 16
NW = NC * NS
B_PER_W = BATCH // NW
CHUNK = 128
N_CHUNKS = B_PER_W // CHUNK


def _gather_body(user_table, movie_table, uids, mids, u_out, m_out,
                 uidx, midx, urows, mrows, sem):
    wid = lax.axis_index("s") * NC + lax.axis_index("c")
    pltpu.sync_copy(uids.at[wid], uidx)
    pltpu.sync_copy(mids.at[wid], midx)
    copies = []
    for j in range(N_CHUNKS):
        copies.append(pltpu.async_copy(user_table.at[uidx.at[j]], urows.at[j], sem))
        copies.append(pltpu.async_copy(movie_table.at[midx.at[j]], mrows.at[j], sem))
    for c in copies:
        c.wait()
    pltpu.sync_copy(urows, u_out.at[wid])
    pltpu.sync_copy(mrows, m_out.at[wid])


_gather = functools.partial(
    pl.kernel,
    out_type=(
        jax.ShapeDtypeStruct((NW, N_CHUNKS, CHUNK, EMBED), jnp.float32),
        jax.ShapeDtypeStruct((NW, N_CHUNKS, CHUNK, EMBED), jnp.float32),
    ),
    mesh=plsc.VectorSubcoreMesh(core_axis_name="c", subcore_axis_name="s"),
    scratch_types=[
        pltpu.VMEM((N_CHUNKS, CHUNK), jnp.int32),
        pltpu.VMEM((N_CHUNKS, CHUNK), jnp.int32),
        pltpu.VMEM((N_CHUNKS, CHUNK, EMBED), jnp.float32),
        pltpu.VMEM((N_CHUNKS, CHUNK, EMBED), jnp.float32),
        pltpu.SemaphoreType.DMA,
    ],
    compiler_params=pltpu.CompilerParams(use_tc_tiling_on_sc=False),
)(_gather_body)


BS = 2048


def _mlp_body(u_ref, m_ref, w1u_ref, w1m_ref, b1_ref, w2_ref, b2_ref, out_ref):
    h = jnp.dot(u_ref[...], w1u_ref[...], preferred_element_type=jnp.float32)
    h = h + jnp.dot(m_ref[...], w1m_ref[...], preferred_element_type=jnp.float32)
    h = jnp.maximum(h + b1_ref[...], 0.0)
    out_ref[...] = jnp.sum(h * w2_ref[...], axis=1) + b2_ref[0, 0]


def _mlp(u, m, w1u, w1m, b1, w2, b2):
    grid = (BATCH // BS,)
    return pl.pallas_call(
        _mlp_body,
        grid=grid,
        in_specs=[
            pl.BlockSpec((BS, EMBED), lambda i: (i, 0)),
            pl.BlockSpec((BS, EMBED), lambda i: (i, 0)),
            pl.BlockSpec((EMBED, HIDDEN), lambda i: (0, 0)),
            pl.BlockSpec((EMBED, HIDDEN), lambda i: (0, 0)),
            pl.BlockSpec((1, HIDDEN), lambda i: (0, 0)),
            pl.BlockSpec((1, HIDDEN), lambda i: (0, 0)),
            pl.BlockSpec(memory_space=pltpu.SMEM),
        ],
        out_specs=pl.BlockSpec((BS,), lambda i: (i,)),
        out_shape=jax.ShapeDtypeStruct((BATCH,), jnp.float32),
    )(u, m, w1u, w1m, b1, w2, b2)


def kernel(user_ids, movie_ids, user_table, movie_table, W1, b1, W2, b2):
    uids = user_ids.astype(jnp.int32).reshape(NW, N_CHUNKS, CHUNK)
    mids = movie_ids.astype(jnp.int32).reshape(NW, N_CHUNKS, CHUNK)
    u_out, m_out = _gather(user_table, movie_table, uids, mids)
    u = u_out.reshape(BATCH, EMBED)
    m = m_out.reshape(BATCH, EMBED)
    w1t = W1.T
    out = _mlp(u, m, w1t[:EMBED], w1t[EMBED:], b1.reshape(1, HIDDEN),
               W2.reshape(1, HIDDEN), b2.reshape(1, 1))
    return out
